# baseline (speedup 1.0000x reference)
import functools

import jax
import jax.numpy as jnp
from jax import lax
from jax.experimental import pallas as pl
from jax.experimental.pallas import tpu as pltpu
from jax.experimental.pallas import tpu_sc as plsc

EMB = 16
BATCH = 16384
CHUNK = 16
LANE = 128


@functools.lru_cache(maxsize=None)
def _make_lookup_kernel(batch: int, emb: int):
    info = plsc.get_sparse_core_info()
    nw = info.num_cores * info.num_subcores
    bpw = batch // nw
    n_chunks = bpw // CHUNK
    mesh = plsc.VectorSubcoreMesh(core_axis_name="c", subcore_axis_name="s")

    @functools.partial(
        pl.kernel,
        mesh=mesh,
        compiler_params=pltpu.CompilerParams(needs_layout_passes=False),
        out_type=[
            jax.ShapeDtypeStruct((emb, batch), jnp.float32),
            jax.ShapeDtypeStruct((emb, batch), jnp.float32),
        ],
        scratch_types=[
            pltpu.VMEM((bpw,), jnp.int32),
            pltpu.VMEM((bpw,), jnp.int32),
            pltpu.VMEM((CHUNK, emb, LANE), jnp.float32),
            pltpu.VMEM((CHUNK, emb, LANE), jnp.float32),
            pltpu.VMEM((emb, bpw), jnp.float32),
            pltpu.VMEM((emb, bpw), jnp.float32),
            pltpu.SemaphoreType.DMA,
            pltpu.SemaphoreType.DMA,
        ],
    )
    def lookup(users_hbm, items_hbm, ut_hbm, it_hbm, ou_hbm, oi_hbm,
               idx_u, idx_i, ring0, ring1, out_u, out_i, sem0, sem1):
        wid = lax.axis_index("s") * info.num_cores + lax.axis_index("c")
        base = wid * bpw
        pltpu.sync_copy(users_hbm.at[pl.ds(base, bpw)], idx_u)
        pltpu.sync_copy(items_hbm.at[pl.ds(base, bpw)], idx_i)
        iota = lax.iota(jnp.int32, 16)

        def fetch(c, idx_ref, tab_hbm, ring, sem):
            off = pl.multiple_of(c * CHUNK, CHUNK)
            vec = idx_ref[pl.ds(off, CHUNK)]
            for lane in range(CHUNK):
                blk = pl.multiple_of((vec[lane] >> 7) * LANE, LANE)
                pltpu.async_copy(tab_hbm.at[pl.ds(0, 8), pl.ds(blk, LANE)],
                                 ring.at[lane, pl.ds(0, 8)], sem)
                pltpu.async_copy(tab_hbm.at[pl.ds(8, 8), pl.ds(blk, LANE)],
                                 ring.at[lane, pl.ds(8, 8)], sem)

        def extract(c, idx_ref, tab_hbm, ring, sem, out):
            off = pl.multiple_of(c * CHUNK, CHUNK)
            vec = idx_ref[pl.ds(off, CHUNK)]
            for lane in range(CHUNK):
                pltpu.make_async_copy(tab_hbm.at[:, pl.ds(0, LANE)],
                                      ring.at[lane], sem).wait()
            for lane in range(CHUNK):
                col = jnp.broadcast_to(vec[lane] & (LANE - 1), (16,))
                dst = jnp.broadcast_to(off + lane, (16,))
                val = plsc.load_gather(ring.at[lane], [iota, col])
                plsc.store_scatter(out, [iota, dst], val)

        fetch(0, idx_u, ut_hbm, ring0, sem0)

        @pl.loop(0, 2 * n_chunks - 1)
        def pipeline(vc):
            c = vc >> 1
            even = (vc & 1) == 0

            @pl.when(even)
            def _():
                fetch(c, idx_i, it_hbm, ring1, sem1)
                extract(c, idx_u, ut_hbm, ring0, sem0, out_u)

            @pl.when(jnp.logical_not(even))
            def _():
                fetch(c + 1, idx_u, ut_hbm, ring0, sem0)
                extract(c, idx_i, it_hbm, ring1, sem1, out_i)

        extract(n_chunks - 1, idx_i, it_hbm, ring1, sem1, out_i)

        pltpu.sync_copy(out_u, ou_hbm.at[:, pl.ds(base, bpw)])
        pltpu.sync_copy(out_i, oi_hbm.at[:, pl.ds(base, bpw)])

    return lookup


def kernel(users, items, user_emb, item_emb):
    batch, = users.shape
    emb = user_emb.shape[1]
    lookup = _make_lookup_kernel(batch, emb)
    ou, oi = lookup(users.astype(jnp.int32), items.astype(jnp.int32),
                    user_emb.T, item_emb.T)
    return (ou.T, oi.T)


# speedup vs baseline: 1.0055x; 1.0055x over previous
import functools

import jax
import jax.numpy as jnp
from jax import lax
from jax.experimental import pallas as pl
from jax.experimental.pallas import tpu as pltpu
from jax.experimental.pallas import tpu_sc as plsc

EMB = 16
BATCH = 16384
CHUNK = 16
LANE = 128


@functools.lru_cache(maxsize=None)
def _make_lookup_kernel(batch: int, emb: int):
    info = plsc.get_sparse_core_info()
    nw = info.num_cores * info.num_subcores
    bpw = batch // nw
    n_chunks = bpw // CHUNK
    mesh = plsc.VectorSubcoreMesh(core_axis_name="c", subcore_axis_name="s")

    @functools.partial(
        pl.kernel,
        mesh=mesh,
        compiler_params=pltpu.CompilerParams(needs_layout_passes=False),
        out_type=[
            jax.ShapeDtypeStruct((emb, batch), jnp.float32),
            jax.ShapeDtypeStruct((emb, batch), jnp.float32),
        ],
        scratch_types=[
            pltpu.VMEM((bpw,), jnp.int32),
            pltpu.VMEM((bpw,), jnp.int32),
            pltpu.VMEM((CHUNK, emb, LANE), jnp.float32),
            pltpu.VMEM((CHUNK, emb, LANE), jnp.float32),
            pltpu.VMEM((emb, bpw), jnp.float32),
            pltpu.VMEM((emb, bpw), jnp.float32),
            pltpu.SemaphoreType.DMA,
            pltpu.SemaphoreType.DMA,
        ],
    )
    def lookup(users_hbm, items_hbm, ut_hbm, it_hbm, ou_hbm, oi_hbm,
               idx_u, idx_i, ring0, ring1, out_u, out_i, sem0, sem1):
        wid = lax.axis_index("s") * info.num_cores + lax.axis_index("c")
        base = wid * bpw
        pltpu.sync_copy(users_hbm.at[pl.ds(base, bpw)], idx_u)
        pltpu.sync_copy(items_hbm.at[pl.ds(base, bpw)], idx_i)
        iota = lax.iota(jnp.int32, 16)

        def fetch(c, idx_ref, tab_hbm, ring, sem):
            off = pl.multiple_of(c * CHUNK, CHUNK)
            vec = idx_ref[pl.ds(off, CHUNK)]
            for lane in range(CHUNK):
                blk = pl.multiple_of((vec[lane] >> 7) * LANE, LANE)
                pltpu.async_copy(tab_hbm.at[:, pl.ds(blk, LANE)],
                                 ring.at[lane], sem)

        def extract(c, idx_ref, tab_hbm, ring, sem, out):
            off = pl.multiple_of(c * CHUNK, CHUNK)
            vec = idx_ref[pl.ds(off, CHUNK)]
            for lane in range(CHUNK):
                pltpu.make_async_copy(tab_hbm.at[:, pl.ds(0, LANE)],
                                      ring.at[lane], sem).wait()
            for lane in range(CHUNK):
                col = jnp.broadcast_to(vec[lane] & (LANE - 1), (16,))
                dst = jnp.broadcast_to(off + lane, (16,))
                val = plsc.load_gather(ring.at[lane], [iota, col])
                plsc.store_scatter(out, [iota, dst], val)

        fetch(0, idx_u, ut_hbm, ring0, sem0)

        @pl.loop(0, 2 * n_chunks - 1)
        def pipeline(vc):
            c = vc >> 1
            even = (vc & 1) == 0

            @pl.when(even)
            def _():
                fetch(c, idx_i, it_hbm, ring1, sem1)
                extract(c, idx_u, ut_hbm, ring0, sem0, out_u)

            @pl.when(jnp.logical_not(even))
            def _():
                fetch(c + 1, idx_u, ut_hbm, ring0, sem0)
                extract(c, idx_i, it_hbm, ring1, sem1, out_i)

        extract(n_chunks - 1, idx_i, it_hbm, ring1, sem1, out_i)

        pltpu.sync_copy(out_u, ou_hbm.at[:, pl.ds(base, bpw)])
        pltpu.sync_copy(out_i, oi_hbm.at[:, pl.ds(base, bpw)])

    return lookup


def kernel(users, items, user_emb, item_emb):
    batch, = users.shape
    emb = user_emb.shape[1]
    lookup = _make_lookup_kernel(batch, emb)
    ou, oi = lookup(users.astype(jnp.int32), items.astype(jnp.int32),
                    user_emb.T, item_emb.T)
    return (ou.T, oi.T)


# speedup vs baseline: 1.0812x; 1.0752x over previous
import functools

import jax
import jax.numpy as jnp
from jax import lax
from jax.experimental import pallas as pl
from jax.experimental.pallas import tpu as pltpu
from jax.experimental.pallas import tpu_sc as plsc

EMB = 16
BATCH = 16384
CHUNK = 16
LANE = 128
NRING = 3


@functools.lru_cache(maxsize=None)
def _make_lookup_kernel(batch: int, emb: int):
    info = plsc.get_sparse_core_info()
    nw = info.num_cores * info.num_subcores
    bpw = batch // nw
    n_chunks = bpw // CHUNK
    nvc = 2 * n_chunks
    mesh = plsc.VectorSubcoreMesh(core_axis_name="c", subcore_axis_name="s")

    @functools.partial(
        pl.kernel,
        mesh=mesh,
        compiler_params=pltpu.CompilerParams(needs_layout_passes=False),
        out_type=[
            jax.ShapeDtypeStruct((emb, batch), jnp.float32),
            jax.ShapeDtypeStruct((emb, batch), jnp.float32),
        ],
        scratch_types=[
            pltpu.VMEM((bpw,), jnp.int32),
            pltpu.VMEM((bpw,), jnp.int32),
            pltpu.VMEM((NRING, CHUNK, emb, LANE), jnp.float32),
            pltpu.VMEM((emb, bpw), jnp.float32),
            pltpu.VMEM((emb, bpw), jnp.float32),
            pltpu.SemaphoreType.DMA,
            pltpu.SemaphoreType.DMA,
            pltpu.SemaphoreType.DMA,
            pltpu.SemaphoreType.DMA,
        ],
    )
    def lookup(users_hbm, items_hbm, ut_hbm, it_hbm, ou_hbm, oi_hbm,
               idx_u, idx_i, rings, out_u, out_i,
               sem0, sem1, sem2, sem_io):
        wid = lax.axis_index("s") * info.num_cores + lax.axis_index("c")
        base = wid * bpw
        cu = pltpu.async_copy(users_hbm.at[pl.ds(base, bpw)], idx_u, sem_io)
        ci = pltpu.async_copy(items_hbm.at[pl.ds(base, bpw)], idx_i, sem_io)
        cu.wait()
        ci.wait()
        iota = lax.iota(jnp.int32, 16)
        sems = (sem0, sem1, sem2)

        def fetch(c, idx_ref, tab_hbm, slot, sem):
            off = pl.multiple_of(c * CHUNK, CHUNK)
            vec = idx_ref[pl.ds(off, CHUNK)]
            for lane in range(CHUNK):
                blk = pl.multiple_of((vec[lane] >> 7) * LANE, LANE)
                pltpu.async_copy(tab_hbm.at[:, pl.ds(blk, LANE)],
                                 rings.at[slot, lane], sem)

        def extract(c, idx_ref, tab_hbm, slot, sem, out):
            off = pl.multiple_of(c * CHUNK, CHUNK)
            vec = idx_ref[pl.ds(off, CHUNK)]
            for lane in range(CHUNK):
                pltpu.make_async_copy(tab_hbm.at[:, pl.ds(0, LANE)],
                                      rings.at[slot, lane], sem).wait()
            for lane in range(CHUNK):
                col = jnp.broadcast_to(vec[lane] & (LANE - 1), (16,))
                dst = jnp.broadcast_to(off + lane, (16,))
                val = plsc.load_gather(rings.at[slot, lane], [iota, col])
                plsc.store_scatter(out, [iota, dst], val)

        def vchunk(vc):
            if vc % 2 == 0:
                return vc // 2, idx_u, ut_hbm, out_u
            return vc // 2, idx_i, it_hbm, out_i

        for vc in range(NRING - 1):
            c, idx_ref, tab, _ = vchunk(vc)
            fetch(c, idx_ref, tab, vc % NRING, sems[vc % NRING])

        period = 2 * NRING
        n_steady = nvc - (NRING - 1)

        @pl.loop(0, n_steady // period)
        def pipeline(g):
            vc0 = g * period
            for k in range(period):
                fc, fidx, ftab, _ = vchunk(k + NRING - 1)
                fcd = vc0 // 2 + fc
                fslot = (k + NRING - 1) % NRING
                fetch(fcd, fidx, ftab, fslot, sems[fslot])
                ec, eidx, etab, eout = vchunk(k)
                ecd = vc0 // 2 + ec
                eslot = k % NRING
                extract(ecd, eidx, etab, eslot, sems[eslot], eout)

        rem = n_steady % period
        tail_start = nvc - (NRING - 1) - rem
        for k in range(rem):
            vc = tail_start + k
            c, idx_ref, tab, out = vchunk(vc)
            fvc = vc + NRING - 1
            if fvc < nvc:
                fc, fidx, ftab, _ = vchunk(fvc)
                fetch(fc, fidx, ftab, fvc % NRING, sems[fvc % NRING])
            extract(c, idx_ref, tab, vc % NRING, sems[vc % NRING], out)
        for vc in range(nvc - (NRING - 1), nvc):
            c, idx_ref, tab, out = vchunk(vc)
            extract(c, idx_ref, tab, vc % NRING, sems[vc % NRING], out)

        co = pltpu.async_copy(out_u, ou_hbm.at[:, pl.ds(base, bpw)], sem_io)
        cp = pltpu.async_copy(out_i, oi_hbm.at[:, pl.ds(base, bpw)], sem_io)
        co.wait()
        cp.wait()

    return lookup


def kernel(users, items, user_emb, item_emb):
    batch, = users.shape
    emb = user_emb.shape[1]
    lookup = _make_lookup_kernel(batch, emb)
    ou, oi = lookup(users.astype(jnp.int32), items.astype(jnp.int32),
                    user_emb.T, item_emb.T)
    return (ou.T, oi.T)
